# baseline (speedup 1.0000x reference)
import functools

import jax
import jax.numpy as jnp
from jax import lax
from jax.experimental import pallas as pl
from jax.experimental.pallas import tpu as pltpu
from jax.experimental.pallas import tpu_sc as plsc

_L = 128
_C = 128
_NTILES = 32
_NSUB = 16


def _cdiv(a, b):
    return (a + b - 1) // b


def _make_deg_kernel(np_pad, chunks_per_tile, rows_per_tile):
    mesh = plsc.VectorSubcoreMesh(core_axis_name="c", subcore_axis_name="s")

    @functools.partial(
        pl.kernel,
        out_type=jax.ShapeDtypeStruct((2, np_pad), jnp.float32),
        mesh=mesh,
        scratch_types=[
            pltpu.VMEM_SHARED((np_pad,), jnp.float32),
            pltpu.VMEM((chunks_per_tile, _C), jnp.int32),
            pltpu.VMEM((_C,), jnp.float32),
        ],
    )
    def deg_kernel(dst_hbm, ones_hbm, zeros_hbm, out_hbm, dacc, dstb, onesb):
        cid = lax.axis_index("c")
        sid = lax.axis_index("s")
        wid = cid * _NSUB + sid
        pltpu.sync_copy(zeros_hbm.at[pl.ds(0, rows_per_tile)],
                        dacc.at[pl.ds(sid * rows_per_tile, rows_per_tile)])
        pltpu.sync_copy(ones_hbm, onesb)
        pltpu.sync_copy(dst_hbm.at[pl.ds(wid * chunks_per_tile, chunks_per_tile)],
                        dstb)
        plsc.subcore_barrier()

        def body(j, carry):
            pltpu.sync_copy(onesb, dacc.at[dstb.at[j]], add=True)
            return carry

        lax.fori_loop(0, chunks_per_tile, body, 0, unroll=False)
        plsc.subcore_barrier()
        pltpu.sync_copy(dacc.at[pl.ds(sid * rows_per_tile, rows_per_tile)],
                        out_hbm.at[cid, pl.ds(sid * rows_per_tile, rows_per_tile)])

    return deg_kernel


_CA = 24
_GRP = 8


def _make_agg_kernel(np_agg, chunks_per_tile, rows_per_tile):
    half_np = np_agg // 2
    stage_rows = np_agg // _NSUB
    ngroups = chunks_per_tile // _GRP
    mesh = plsc.VectorSubcoreMesh(core_axis_name="c", subcore_axis_name="s")

    @functools.partial(
        pl.kernel,
        out_type=jax.ShapeDtypeStruct((2, np_agg, _L), jnp.float32),
        mesh=mesh,
        scratch_types=[
            pltpu.VMEM_SHARED((np_agg, _L), jnp.float32),
            pltpu.VMEM_SHARED((half_np, _L), jnp.float32),
            pltpu.VMEM((_GRP, _CA), jnp.int32),
            pltpu.VMEM((_GRP, _CA), jnp.int32),
            pltpu.VMEM((_CA, _L), jnp.float32),
            pltpu.VMEM((_CA, _L), jnp.float32),
            pltpu.SemaphoreType.DMA,
            pltpu.SemaphoreType.DMA,
        ],
    )
    def agg_kernel(t_hbm, gs0_hbm, gd0_hbm, gs1_hbm, gd1_hbm, zeros_hbm,
                   out_hbm, acc, ts, srcb, dstb, r0, r1, s0, s1):
        cid = lax.axis_index("c")
        sid = lax.axis_index("s")
        wid = cid * _NSUB + sid
        rows = [r0, r1]
        sems = [s0, s1]

        pltpu.sync_copy(zeros_hbm.at[pl.ds(0, rows_per_tile)],
                        acc.at[pl.ds(sid * rows_per_tile, rows_per_tile)])

        for p, (gs_hbm, gd_hbm) in enumerate(((gs0_hbm, gd0_hbm),
                                              (gs1_hbm, gd1_hbm))):
            @pl.when(sid < 8)
            def _():
                pltpu.sync_copy(
                    t_hbm.at[pl.ds(p * half_np + sid * stage_rows, stage_rows)],
                    ts.at[pl.ds(sid * stage_rows, stage_rows)])

            plsc.subcore_barrier()

            def group(g, carry):
                base = wid * chunks_per_tile + g * _GRP
                pltpu.sync_copy(gs_hbm.at[pl.ds(base, _GRP)], srcb)
                pltpu.sync_copy(gd_hbm.at[pl.ds(base, _GRP)], dstb)
                pltpu.make_async_copy(ts.at[srcb.at[0]], r0, s0).start()

                def body(k, c2):
                    for b in range(2):
                        j = 2 * k + b
                        pltpu.make_async_copy(
                            ts.at[srcb.at[j]], rows[b], sems[b]).wait()
                        pltpu.make_async_copy(
                            ts.at[srcb.at[j + 1]], rows[1 - b],
                            sems[1 - b]).start()
                        pltpu.sync_copy(rows[b], acc.at[dstb.at[j]], add=True)
                    return c2

                lax.fori_loop(0, _GRP // 2 - 1, body, 0, unroll=False)
                j = _GRP - 2
                pltpu.make_async_copy(ts.at[srcb.at[j]], r0, s0).wait()
                pltpu.make_async_copy(ts.at[srcb.at[j + 1]], r1, s1).start()
                pltpu.sync_copy(r0, acc.at[dstb.at[j]], add=True)
                pltpu.make_async_copy(ts.at[srcb.at[j + 1]], r1, s1).wait()
                pltpu.sync_copy(r1, acc.at[dstb.at[j + 1]], add=True)
                return carry

            lax.fori_loop(0, ngroups, group, 0, unroll=False)
            plsc.subcore_barrier()

        pltpu.sync_copy(acc.at[pl.ds(sid * rows_per_tile, rows_per_tile)],
                        out_hbm.at[cid, pl.ds(sid * rows_per_tile, rows_per_tile)])

    return agg_kernel


_RB = 1000


def _tc_first(degp, x, w):
    n = x.shape[0]
    grid = n // _RB

    def body(degp_ref, x_ref, w_ref, t_ref, dinv_ref):
        deg = degp_ref[0] + degp_ref[1] + 1.0
        dinv = lax.rsqrt(deg)
        h = jnp.dot(x_ref[...], w_ref[...],
                    preferred_element_type=jnp.float32)
        t_ref[...] = h * dinv
        dinv_ref[...] = dinv

    return pl.pallas_call(
        body,
        grid=(grid,),
        in_specs=[
            pl.BlockSpec((2, _RB, 1), lambda i: (0, i, 0)),
            pl.BlockSpec((_RB, _L), lambda i: (i, 0)),
            pl.BlockSpec((_L, _L), lambda i: (0, 0)),
        ],
        out_specs=[
            pl.BlockSpec((_RB, _L), lambda i: (i, 0)),
            pl.BlockSpec((_RB, 1), lambda i: (i, 0)),
        ],
        out_shape=[
            jax.ShapeDtypeStruct((n, _L), jnp.float32),
            jax.ShapeDtypeStruct((n, 1), jnp.float32),
        ],
    )(degp, x, w)


def _tc_mid(sp, t, dinv, b, w):
    n = t.shape[0]
    grid = n // _RB

    def body(sp_ref, t_ref, dinv_ref, b_ref, w_ref, t2_ref):
        s = sp_ref[0] + sp_ref[1] + t_ref[...]
        y = jnp.maximum(s * dinv_ref[...] + b_ref[...], 0.0)
        h2 = jnp.dot(y, w_ref[...], preferred_element_type=jnp.float32)
        t2_ref[...] = h2 * dinv_ref[...]

    return pl.pallas_call(
        body,
        grid=(grid,),
        in_specs=[
            pl.BlockSpec((2, _RB, _L), lambda i: (0, i, 0)),
            pl.BlockSpec((_RB, _L), lambda i: (i, 0)),
            pl.BlockSpec((_RB, 1), lambda i: (i, 0)),
            pl.BlockSpec((1, _L), lambda i: (0, 0)),
            pl.BlockSpec((_L, _L), lambda i: (0, 0)),
        ],
        out_specs=pl.BlockSpec((_RB, _L), lambda i: (i, 0)),
        out_shape=jax.ShapeDtypeStruct((n, _L), jnp.float32),
    )(sp, t, dinv, b, w)


def _tc_last(sp, t, dinv, b):
    n = t.shape[0]
    grid = n // _RB

    def body(sp_ref, t_ref, dinv_ref, b_ref, out_ref):
        s = sp_ref[0] + sp_ref[1] + t_ref[...]
        out_ref[...] = jnp.maximum(s * dinv_ref[...] + b_ref[...], 0.0)

    return pl.pallas_call(
        body,
        grid=(grid,),
        in_specs=[
            pl.BlockSpec((2, _RB, _L), lambda i: (0, i, 0)),
            pl.BlockSpec((_RB, _L), lambda i: (i, 0)),
            pl.BlockSpec((_RB, 1), lambda i: (i, 0)),
            pl.BlockSpec((1, _L), lambda i: (0, 0)),
        ],
        out_specs=pl.BlockSpec((_RB, _L), lambda i: (i, 0)),
        out_shape=jax.ShapeDtypeStruct((n, _L), jnp.float32),
    )(sp, t, dinv, b)


def kernel(x, edge_index, W1, b1, W2, b2):
    n, d = x.shape
    e = edge_index.shape[1]

    np_deg = _cdiv(n + 1, _NSUB * 128) * _NSUB * 128
    rows_deg = np_deg // _NSUB
    chunks_deg = _cdiv(e, _NTILES * _C * 8) * 8
    ep_deg = chunks_deg * _NTILES * _C

    np_agg = _cdiv(n + 1, _NSUB * 8) * _NSUB * 8
    rows_agg = np_agg // _NSUB
    half_np = np_agg // 2
    chunks_agg = _cdiv(e, _NTILES * _CA * 8) * 8
    ep_agg = chunks_agg * _NTILES * _CA

    src = edge_index[0]
    dst = edge_index[1]

    pad_d = ep_deg - e
    dst_p = jnp.concatenate(
        [dst, n + jnp.arange(pad_d, dtype=jnp.int32) % (np_deg - n)]
    ).reshape(ep_deg // _C, _C)

    pad_a = ep_agg - e
    pos = jnp.arange(ep_agg, dtype=jnp.int32)
    srcf = jnp.concatenate([src, jnp.zeros((pad_a,), jnp.int32)])
    dstf = jnp.concatenate([dst, jnp.zeros((pad_a,), jnp.int32)])
    valid = pos < e
    scr = n + pos % (np_agg - n)
    dmy = pos % half_np
    gs, gd = [], []
    for p in range(2):
        inp = valid & (srcf >= p * half_np) & (srcf < (p + 1) * half_np)
        gs.append(jnp.where(inp, srcf - p * half_np, dmy)
                  .reshape(ep_agg // _CA, _CA))
        gd.append(jnp.where(inp, dstf, scr).reshape(ep_agg // _CA, _CA))

    zeros2d = jnp.zeros((rows_agg, _L), jnp.float32)
    zeros1d = jnp.zeros((rows_deg,), jnp.float32)
    ones1d = jnp.ones((_C,), jnp.float32)

    deg_k = _make_deg_kernel(np_deg, chunks_deg, rows_deg)
    agg_k = _make_agg_kernel(np_agg, chunks_agg, rows_agg)

    degp = deg_k(dst_p, ones1d, zeros1d)
    degp3 = degp[:, :, None]

    t1, dinv = _tc_first(degp3, x, W1)
    t1p = jnp.pad(t1, ((0, np_agg - n), (0, 0)))
    sp1 = agg_k(t1p, gs[0], gd[0], gs[1], gd[1], zeros2d)
    t2 = _tc_mid(sp1, t1, dinv, b1.reshape(1, _L), W2)
    t2p = jnp.pad(t2, ((0, np_agg - n), (0, 0)))
    sp2 = agg_k(t2p, gs[0], gd[0], gs[1], gd[1], zeros2d)
    return _tc_last(sp2, t2, dinv, b2.reshape(1, _L))
